# baseline (speedup 1.0000x reference)
import functools

import jax
import jax.numpy as jnp
from jax import lax
from jax.experimental import pallas as pl
from jax.experimental.pallas import tpu as pltpu
from jax.experimental.pallas import tpu_sc as plsc

_NC = 2
_NS = 16
_L = 16

_BIG_I32 = 2**30
_INF = float("inf")


def _tc_prep_body(xs_ref, ws_ref, bs_ref, g_ref, be_ref, wm_ref, bm_ref,
                  z_ref):
    h = jnp.dot(xs_ref[...], ws_ref[...], preferred_element_type=jnp.float32)
    h = h + bs_ref[...]
    mean = jnp.mean(h, axis=0, keepdims=True)
    var = jnp.mean((h - mean) ** 2, axis=0, keepdims=True)
    h = (h - mean) / jnp.sqrt(var + 1e-5) * g_ref[...] + be_ref[...]
    h = jnp.maximum(h, 0.0)
    z_ref[...] = (
        jnp.dot(h, wm_ref[...], preferred_element_type=jnp.float32)
        + bm_ref[...]
    )


def _tc_knn_body(d_out, p_ref, qt_ref, qq_ref, sq_ref, w_ref):
    mm = jnp.dot(p_ref[0], qt_ref[0], preferred_element_type=jnp.float32)
    d = -2.0 * mm
    d = d + qq_ref[0]
    d = d + sq_ref[0]
    d = jnp.clip(d, 1e-16, None)

    n_sub, tq = d.shape
    iota = lax.broadcasted_iota(jnp.int32, (n_sub, tq), 0)
    for r in range(3):
        mn = jnp.min(d, axis=0, keepdims=True)
        ii = jnp.min(jnp.where(d == mn, iota, _BIG_I32), axis=0,
                     keepdims=True)
        w_ref[0, r, :] = (1.0 / mn)[0]
        w_ref[0, 3 + r, :] = (ii * (d_out + 1)).astype(jnp.float32)[0]
        if r < 2:
            d = jnp.where(iota == ii, _INF, d)
    w_ref[0, 6, :] = jnp.zeros((tq,), jnp.float32)
    w_ref[0, 7, :] = jnp.zeros((tq,), jnp.float32)


def _sc_interp_body(n_sub, n_per_w, w_hbm, z_hbm, out_hbm, wbuf, z_v, out_v):
    d_out = 2 * _L
    c = lax.axis_index("c")
    s = lax.axis_index("s")
    qbase = (c * _NS + s) * n_per_w

    pltpu.sync_copy(w_hbm.at[:, pl.ds(qbase, n_per_w)], wbuf)
    pltpu.sync_copy(z_hbm, z_v)

    def group_body(g, carry):
        o = pl.multiple_of(g * _L, _L)
        w1 = wbuf[0, pl.ds(o, _L)]
        w2 = wbuf[1, pl.ds(o, _L)]
        w3 = wbuf[2, pl.ds(o, _L)]
        i1 = wbuf[3, pl.ds(o, _L)].astype(jnp.int32)
        i2 = wbuf[4, pl.ds(o, _L)].astype(jnp.int32)
        i3 = wbuf[5, pl.ds(o, _L)].astype(jnp.int32)
        rcp = 1.0 / (w1 + w2 + w3)
        sw1 = w1 * rcp
        sw2 = w2 * rcp
        sw3 = w3 * rcp
        for dim in range(d_out):
            acc = sw1 * plsc.load_gather(z_v, [i1 + dim])
            acc = acc + sw2 * plsc.load_gather(z_v, [i2 + dim])
            acc = acc + sw3 * plsc.load_gather(z_v, [i3 + dim])
            out_v[dim, pl.ds(o, _L)] = acc
        return carry

    lax.fori_loop(0, n_per_w // _L, group_body, 0)

    pltpu.sync_copy(out_v, out_hbm.at[:, pl.ds(qbase, n_per_w)])


def kernel(x, x_sub, pos, pos_sub, W_sub, b_sub, gamma, beta, W_mlp, b_mlp):
    B, N_sub, d_in = x_sub.shape
    d_out = W_sub.shape[1]
    _, N, d_p = pos.shape
    assert d_p == 3 and B == _NC and N % (_NC * _NS * _L) == 0
    assert N_sub % _L == 0 and d_out == 2 * _L
    n_per_w = N // (_NC * _NS)

    xs2 = x_sub.reshape(B * N_sub, d_in)
    z = pl.pallas_call(
        _tc_prep_body,
        out_shape=jax.ShapeDtypeStruct((B * N_sub, d_out), jnp.float32),
    )(
        xs2, W_sub,
        b_sub.reshape(1, d_out), gamma.reshape(1, d_out),
        beta.reshape(1, d_out), W_mlp, b_mlp.reshape(1, d_out),
    )

    qT = jnp.transpose(pos, (0, 2, 1))
    qq = jnp.sum(pos ** 2, axis=-1)[:, None, :]
    sq = jnp.sum(pos_sub ** 2, axis=-1)[..., None]
    z_pad = jnp.pad(z.reshape(B, N_sub, d_out), ((0, 0), (0, 0), (0, 1)))
    TQ = 1024

    mesh = plsc.VectorSubcoreMesh(core_axis_name="c", subcore_axis_name="s")
    outs = []
    for b in range(B):
        wout_b = pl.pallas_call(
            functools.partial(_tc_knn_body, d_out),
            grid=(1, N // TQ),
            in_specs=[
                pl.BlockSpec((1, N_sub, d_p), lambda c, i: (c, 0, 0)),
                pl.BlockSpec((1, d_p, TQ), lambda c, i: (c, 0, i)),
                pl.BlockSpec((1, 1, TQ), lambda c, i: (c, 0, i)),
                pl.BlockSpec((1, N_sub, 1), lambda c, i: (c, 0, 0)),
            ],
            out_specs=pl.BlockSpec((1, 8, TQ), lambda c, i: (c, 0, i)),
            out_shape=jax.ShapeDtypeStruct((1, 8, N), jnp.float32),
        )(pos_sub[b:b + 1], qT[b:b + 1], qq[b:b + 1], sq[b:b + 1])

        out_b = pl.kernel(
            functools.partial(_sc_interp_body, N_sub, n_per_w),
            out_type=jax.ShapeDtypeStruct((d_out, N), jnp.float32),
            mesh=mesh,
            compiler_params=pltpu.CompilerParams(
                needs_layout_passes=False, use_tc_tiling_on_sc=False),
            scratch_types=[
                pltpu.VMEM((8, n_per_w), jnp.float32),
                pltpu.VMEM((N_sub * (d_out + 1),), jnp.float32),
                pltpu.VMEM((d_out, n_per_w), jnp.float32),
            ],
        )(wout_b.reshape(8, N), z_pad[b].reshape(-1))
        outs.append(out_b)
    out_soa = jnp.stack(outs)
    return jnp.transpose(out_soa, (0, 2, 1))


# speedup vs baseline: 1.0672x; 1.0672x over previous
import functools

import jax
import jax.numpy as jnp
from jax import lax
from jax.experimental import pallas as pl
from jax.experimental.pallas import tpu as pltpu
from jax.experimental.pallas import tpu_sc as plsc

_NC = 2
_NS = 16
_L = 16

_BIG_I32 = 2**30
_INF = float("inf")


def _tc_prep_body(xs_ref, ws_ref, bs_ref, g_ref, be_ref, wm_ref, bm_ref,
                  z_ref):
    h = jnp.dot(xs_ref[...], ws_ref[...], preferred_element_type=jnp.float32)
    h = h + bs_ref[...]
    mean = jnp.mean(h, axis=0, keepdims=True)
    var = jnp.mean((h - mean) ** 2, axis=0, keepdims=True)
    h = (h - mean) / jnp.sqrt(var + 1e-5) * g_ref[...] + be_ref[...]
    h = jnp.maximum(h, 0.0)
    z_ref[...] = (
        jnp.dot(h, wm_ref[...], preferred_element_type=jnp.float32)
        + bm_ref[...]
    )


def _tc_knn_body(d_out, p_ref, qt_ref, w_ref):
    p = p_ref[0]
    qt = qt_ref[0]
    mm = jnp.dot(p, qt, preferred_element_type=jnp.float32)
    qq = jnp.sum(qt * qt, axis=0, keepdims=True)
    sq = jnp.sum(p * p, axis=1, keepdims=True)
    d = -2.0 * mm
    d = d + qq
    d = d + sq
    d = jnp.clip(d, 1e-16, None)

    n_sub, tq = d.shape
    iota = lax.broadcasted_iota(jnp.int32, (n_sub, tq), 0)
    for r in range(3):
        mn = jnp.min(d, axis=0, keepdims=True)
        ii = jnp.min(jnp.where(d == mn, iota, _BIG_I32), axis=0,
                     keepdims=True)
        w_ref[0, r, :] = (1.0 / mn)[0]
        w_ref[0, 3 + r, :] = (ii * (d_out + 1)).astype(jnp.float32)[0]
        if r < 2:
            d = jnp.where(iota == ii, _INF, d)
    w_ref[0, 6, :] = jnp.zeros((tq,), jnp.float32)
    w_ref[0, 7, :] = jnp.zeros((tq,), jnp.float32)


def _sc_interp_body(n_sub, n_per_w, w_hbm, z_hbm, out_hbm, wbuf, z_v, out_v):
    d_out = 2 * _L
    c = lax.axis_index("c")
    s = lax.axis_index("s")
    qbase = s * n_per_w

    zlen = n_sub * (d_out + 1)
    pltpu.sync_copy(w_hbm.at[c, :, pl.ds(qbase, n_per_w)], wbuf)
    pltpu.sync_copy(z_hbm.at[pl.ds(c * zlen, zlen)], z_v)

    def group_body(g, carry):
        o = pl.multiple_of(g * _L, _L)
        w1 = wbuf[0, pl.ds(o, _L)]
        w2 = wbuf[1, pl.ds(o, _L)]
        w3 = wbuf[2, pl.ds(o, _L)]
        i1 = wbuf[3, pl.ds(o, _L)].astype(jnp.int32)
        i2 = wbuf[4, pl.ds(o, _L)].astype(jnp.int32)
        i3 = wbuf[5, pl.ds(o, _L)].astype(jnp.int32)
        rcp = 1.0 / (w1 + w2 + w3)
        sw1 = w1 * rcp
        sw2 = w2 * rcp
        sw3 = w3 * rcp
        for dim in range(d_out):
            acc = sw1 * plsc.load_gather(z_v, [i1 + dim])
            acc = acc + sw2 * plsc.load_gather(z_v, [i2 + dim])
            acc = acc + sw3 * plsc.load_gather(z_v, [i3 + dim])
            out_v[dim, pl.ds(o, _L)] = acc
        return carry

    lax.fori_loop(0, n_per_w // _L, group_body, 0)

    pltpu.sync_copy(out_v, out_hbm.at[c, :, pl.ds(qbase, n_per_w)])


def kernel(x, x_sub, pos, pos_sub, W_sub, b_sub, gamma, beta, W_mlp, b_mlp):
    B, N_sub, d_in = x_sub.shape
    d_out = W_sub.shape[1]
    _, N, d_p = pos.shape
    assert d_p == 3 and B == _NC and N % (_NS * _L) == 0
    assert N_sub % _L == 0 and d_out == 2 * _L
    n_per_w = N // _NS

    xs2 = x_sub.reshape(B * N_sub, d_in)
    z = pl.pallas_call(
        _tc_prep_body,
        out_shape=jax.ShapeDtypeStruct((B * N_sub, d_out), jnp.float32),
    )(
        xs2, W_sub,
        b_sub.reshape(1, d_out), gamma.reshape(1, d_out),
        beta.reshape(1, d_out), W_mlp, b_mlp.reshape(1, d_out),
    )

    qT = jnp.transpose(pos, (0, 2, 1))
    TQ = 1024
    wout = pl.pallas_call(
        functools.partial(_tc_knn_body, d_out),
        grid=(B, N // TQ),
        in_specs=[
            pl.BlockSpec((1, N_sub, d_p), lambda b, i: (b, 0, 0)),
            pl.BlockSpec((1, d_p, TQ), lambda b, i: (b, 0, i)),
        ],
        out_specs=pl.BlockSpec((1, 8, TQ), lambda b, i: (b, 0, i)),
        out_shape=jax.ShapeDtypeStruct((B, 8, N), jnp.float32),
    )(pos_sub, qT)

    z_pad = jnp.pad(z.reshape(B, N_sub, d_out), ((0, 0), (0, 0), (0, 1)))

    mesh = plsc.VectorSubcoreMesh(core_axis_name="c", subcore_axis_name="s")
    out_soa = pl.kernel(
        functools.partial(_sc_interp_body, N_sub, n_per_w),
        out_type=jax.ShapeDtypeStruct((B, d_out, N), jnp.float32),
        mesh=mesh,
        compiler_params=pltpu.CompilerParams(
            needs_layout_passes=False, use_tc_tiling_on_sc=False),
        scratch_types=[
            pltpu.VMEM((8, n_per_w), jnp.float32),
            pltpu.VMEM((N_sub * (d_out + 1),), jnp.float32),
            pltpu.VMEM((d_out, n_per_w), jnp.float32),
        ],
    )(wout, z_pad.reshape(-1))
    return jnp.transpose(out_soa, (0, 2, 1))
